# baseline (speedup 1.0000x reference)
import jax
import jax.numpy as jnp
from jax.experimental import pallas as pl
from jax.experimental.pallas import tpu as pltpu

TN = 256

_INV_SQRT2 = 0.7071067811865476


def _gelu_exact(v):
    return 0.5 * v * (1.0 + jax.lax.erf(v * _INV_SQRT2))


def _fused_body(t_ref, x_ref, w1a_ref, b1a_ref, w1b_ref, b1b_ref,
                w2a_ref, b2a_ref, w2b_ref, b2b_ref, tt_ref, pos_ref,
                gamma_ref, beta_ref, o_ref):
    f32 = jnp.float32
    d1 = w1a_ref.shape[0]
    d2 = w2a_ref.shape[0]
    x = x_ref[...]

    g1 = _gelu_exact(
        jnp.dot(x[:, :d1], w1a_ref[...], preferred_element_type=f32) + b1a_ref[...])
    h1 = jnp.dot(g1, w1b_ref[...], preferred_element_type=f32) + b1b_ref[...]

    g2 = _gelu_exact(
        jnp.dot(x[:, :d2], w2a_ref[...], preferred_element_type=f32) + b2a_ref[...])
    h2 = jnp.dot(g2, w2b_ref[...], preferred_element_type=f32) + b2b_ref[...]

    m1 = t_ref[...] == 0
    h = jnp.where(m1, h1, h2)
    h = h + jnp.where(m1, tt_ref[0:1, :], tt_ref[1:2, :]) + pos_ref[...]

    mu = jnp.mean(h, axis=-1, keepdims=True)
    c = h - mu
    var = jnp.mean(c * c, axis=-1, keepdims=True)
    o_ref[...] = c * jax.lax.rsqrt(var + 1e-5) * gamma_ref[...] + beta_ref[...]


def kernel(x, token_type_ids, W1a, b1a, W1b, b1b, W2a, b2a, W2b, b2b,
           type_table, pos_table, gamma, beta):
    B, L, Dx = x.shape
    DM = W1a.shape[1]
    N = B * L
    n_tiles = N // TN
    pos_blocks = L // TN

    xf = x.reshape(N, Dx)
    tcol = token_type_ids.reshape(N, 1)

    const = lambda g: (0, 0)
    out = pl.pallas_call(
        _fused_body,
        grid=(n_tiles,),
        in_specs=[
            pl.BlockSpec((TN, 1), lambda g: (g, 0)),
            pl.BlockSpec((TN, Dx), lambda g: (g, 0)),
            pl.BlockSpec(W1a.shape, const),
            pl.BlockSpec((1, DM), const),
            pl.BlockSpec(W1b.shape, const),
            pl.BlockSpec((1, DM), const),
            pl.BlockSpec(W2a.shape, const),
            pl.BlockSpec((1, DM), const),
            pl.BlockSpec(W2b.shape, const),
            pl.BlockSpec((1, DM), const),
            pl.BlockSpec((2, DM), const),
            pl.BlockSpec((TN, DM), lambda g: (g % pos_blocks, 0)),
            pl.BlockSpec((1, DM), const),
            pl.BlockSpec((1, DM), const),
        ],
        out_specs=pl.BlockSpec((TN, DM), lambda g: (g, 0)),
        out_shape=jax.ShapeDtypeStruct((N, DM), jnp.float32),
        compiler_params=pltpu.CompilerParams(
            dimension_semantics=("arbitrary",),
        ),
    )(tcol, xf, W1a, b1a.reshape(1, DM), W1b, b1b.reshape(1, DM),
      W2a, b2a.reshape(1, DM), W2b, b2b.reshape(1, DM),
      type_table, pos_table, gamma.reshape(1, DM), beta.reshape(1, DM))

    return out.reshape(B, L, DM)


# speedup vs baseline: 1.0035x; 1.0035x over previous
import jax
import jax.numpy as jnp
from jax.experimental import pallas as pl
from jax.experimental.pallas import tpu as pltpu

TN = 256

_INV_SQRT2 = 0.7071067811865476


def _gelu_exact(v):
    return 0.5 * v * (1.0 + jax.lax.erf(v * _INV_SQRT2))


def _fused_body(t_ref, x_ref, w1a_ref, b1a_ref, w1b_ref, b1b_ref,
                w2a_ref, b2a_ref, w2b_ref, b2b_ref, tt_ref, pos_ref,
                gamma_ref, beta_ref, o_ref):
    f32 = jnp.float32
    d1 = w1a_ref.shape[0]
    d2 = w2a_ref.shape[0]
    x = x_ref[...]

    g1 = _gelu_exact(
        jnp.dot(x[:, :d1], w1a_ref[...], preferred_element_type=f32) + b1a_ref[...])
    h1 = jnp.dot(g1, w1b_ref[...], preferred_element_type=f32) + b1b_ref[...]

    g2 = _gelu_exact(
        jnp.dot(x[:, :d2], w2a_ref[...], preferred_element_type=f32) + b2a_ref[...])
    h2 = jnp.dot(g2, w2b_ref[...], preferred_element_type=f32) + b2b_ref[...]

    m1 = t_ref[...] == 0
    h = jnp.where(m1, h1, h2)
    h = h + jnp.where(m1, tt_ref[0:1, :], tt_ref[1:2, :]) + pos_ref[...]

    mu = jnp.mean(h, axis=-1, keepdims=True)
    ms = jnp.mean(h * h, axis=-1, keepdims=True)
    r = jax.lax.rsqrt(ms - mu * mu + 1e-5) * gamma_ref[...]
    o_ref[...] = (h - mu) * r + beta_ref[...]


def kernel(x, token_type_ids, W1a, b1a, W1b, b1b, W2a, b2a, W2b, b2b,
           type_table, pos_table, gamma, beta):
    B, L, Dx = x.shape
    DM = W1a.shape[1]
    N = B * L
    n_tiles = N // TN
    pos_blocks = L // TN

    xf = x.reshape(N, Dx)
    tcol = token_type_ids.reshape(N, 1)

    const = lambda g: (0, 0)
    out = pl.pallas_call(
        _fused_body,
        grid=(n_tiles,),
        in_specs=[
            pl.BlockSpec((TN, 1), lambda g: (g, 0)),
            pl.BlockSpec((TN, Dx), lambda g: (g, 0)),
            pl.BlockSpec(W1a.shape, const),
            pl.BlockSpec((1, DM), const),
            pl.BlockSpec(W1b.shape, const),
            pl.BlockSpec((1, DM), const),
            pl.BlockSpec(W2a.shape, const),
            pl.BlockSpec((1, DM), const),
            pl.BlockSpec(W2b.shape, const),
            pl.BlockSpec((1, DM), const),
            pl.BlockSpec((2, DM), const),
            pl.BlockSpec((TN, DM), lambda g: (g % pos_blocks, 0)),
            pl.BlockSpec((1, DM), const),
            pl.BlockSpec((1, DM), const),
        ],
        out_specs=pl.BlockSpec((TN, DM), lambda g: (g, 0)),
        out_shape=jax.ShapeDtypeStruct((N, DM), jnp.float32),
        compiler_params=pltpu.CompilerParams(
            dimension_semantics=("arbitrary",),
        ),
    )(tcol, xf, W1a, b1a.reshape(1, DM), W1b, b1b.reshape(1, DM),
      W2a, b2a.reshape(1, DM), W2b, b2b.reshape(1, DM),
      type_table, pos_table, gamma.reshape(1, DM), beta.reshape(1, DM))

    return out.reshape(B, L, DM)
